# baseline (speedup 1.0000x reference)
import functools

import jax
import jax.numpy as jnp
from jax import lax
from jax.experimental import pallas as pl
from jax.experimental.pallas import tpu as pltpu
from jax.experimental.pallas import tpu_sc as plsc

N = 10000
E = 160000
IN_DIM = 256
HID = 512
OUT_DIM = 128
NUM_LAYERS = 4

N_PAD = 10240
TR = 512
GRID = N_PAD // TR

NQ = 4
QD = HID // NQ
NQC = NQ // 2
NT = 16
EPT = E // NT
CH = 40
NCH = EPT // CH
NBUF = 5
GCH = NBUF * CH
NGRP = NCH // NBUF
RPT = N_PAD // NT
ACC_R = 10112
RPT_A = ACC_R // NT
CNT_W = 128

def _count_body(dst_hbm, cnt_hbm, idx_v, ones_v, acc_sh):
    c = lax.axis_index("c")
    s = lax.axis_index("s")

    zeros16 = jnp.zeros((16,), jnp.float32)

    def _zero_row(i, _):
        for j in range(CNT_W // 16):
            ones_v[i, pl.ds(16 * j, 16)] = zeros16
        return 0

    lax.fori_loop(0, CH, _zero_row, 0)
    for k in range(RPT // CH):
        pltpu.sync_copy(ones_v, acc_sh.at[pl.ds(s * RPT + k * CH, CH)])

    def _one_row(i, _):
        for j in range(CNT_W // 16):
            ones_v[i, pl.ds(16 * j, 16)] = zeros16 + 1.0
        return 0

    lax.fori_loop(0, CH, _one_row, 0)
    plsc.subcore_barrier()

    def _chunk(j, _):
        base = s * EPT + j * CH
        pltpu.sync_copy(dst_hbm.at[pl.ds(base, CH)], idx_v)
        pltpu.sync_copy(ones_v, acc_sh.at[idx_v], add=True)
        return 0

    lax.fori_loop(0, NCH, _chunk, 0)
    plsc.subcore_barrier()

    w = c * NT + s
    rows = N_PAD // (2 * NT)
    pltpu.sync_copy(acc_sh.at[pl.ds(w * rows, rows)],
                    cnt_hbm.at[pl.ds(w * rows, rows)])


def _agg_body(hn_hbm, src_hbm, dst4_hbm, out_hbm, idx0, idx1, dstb0, dstb1,
              b0, b1, b2, b3, b4, acc_sh,
              i0, i1, j0, j1, g0, g1, g2, g3, g4, s0, s1, s2, s3, s4):
    c = lax.axis_index("c")
    s = lax.axis_index("s")
    zeros16 = jnp.zeros((16,), jnp.float32)
    bufs = (b0, b1, b2, b3, b4)
    gsem = (g0, g1, g2, g3, g4)
    ssem = (s0, s1, s2, s3, s4)

    def _idx_load(t2, idxb, dstb, isem, jsem):
        g2 = jnp.where(t2 < NGRP, t2, 0)
        pltpu.async_copy(src_hbm.at[pl.ds(s * EPT + g2 * GCH, GCH)],
                         idxb, isem)
        pltpu.async_copy(dst4_hbm.at[s * NGRP + g2], dstb, jsem)

    def _idx_wait(idxb, dstb, isem, jsem):
        pltpu.make_async_copy(src_hbm.at[pl.ds(0, GCH)], idxb, isem).wait()
        pltpu.make_async_copy(dst4_hbm.at[0], dstb, jsem).wait()

    def _do_group(win, idxb, dstb, gbase):
        for b in range(NBUF):
            @pl.when(gbase > 0)
            def _():
                pltpu.make_async_copy(bufs[b], acc_sh.at[pl.ds(0, CH)],
                                      ssem[b]).wait()

            pltpu.async_copy(win.at[idxb.at[pl.ds(b * CH, CH)]],
                             bufs[b], gsem[b])
        for b in range(NBUF):
            pltpu.make_async_copy(hn_hbm.at[pl.ds(0, CH)], bufs[b],
                                  gsem[b]).wait()
            pltpu.async_copy(bufs[b], acc_sh.at[dstb.at[b]],
                             ssem[b], add=True)

    for qi in range(NQC):
        qstart = (NQC * c + qi) * N_PAD
        win = hn_hbm.at[pl.ds(qstart, N_PAD)]

        def _zero_row(i, _):
            for j in range(QD // 16):
                bufs[0][i, pl.ds(16 * j, 16)] = zeros16
            return 0

        lax.fori_loop(0, CH, _zero_row, 0)
        for k in range(RPT_A // CH):
            pltpu.sync_copy(bufs[0], acc_sh.at[pl.ds(s * RPT_A + k * CH, CH)])
        pltpu.sync_copy(
            bufs[0].at[pl.ds(0, RPT_A - (RPT_A // CH) * CH)],
            acc_sh.at[pl.ds(s * RPT_A + (RPT_A // CH) * CH,
                            RPT_A - (RPT_A // CH) * CH)])
        plsc.subcore_barrier()

        _idx_load(0, idx0, dstb0, i0, j0)
        _idx_wait(idx0, dstb0, i0, j0)

        def _pair(t, _):
            _idx_load(2 * t + 1, idx1, dstb1, i1, j1)
            _do_group(win, idx0, dstb0, (2 * t) * NBUF)
            _idx_wait(idx1, dstb1, i1, j1)
            _do_group(win, idx1, dstb1, (2 * t + 1) * NBUF)
            _idx_load(2 * t + 2, idx0, dstb0, i0, j0)
            _idx_wait(idx0, dstb0, i0, j0)
            return 0

        lax.fori_loop(0, NGRP // 2, _pair, 0)
        for b in range(NBUF):
            pltpu.make_async_copy(bufs[b], acc_sh.at[pl.ds(0, CH)],
                                  ssem[b]).wait()
        plsc.subcore_barrier()

        pltpu.sync_copy(acc_sh.at[pl.ds(s * RPT_A, RPT_A)],
                        out_hbm.at[pl.ds((NQC * c + qi) * N_PAD + s * RPT_A,
                                         RPT_A)])


@functools.cache
def _sc_kernels():
    mesh = plsc.VectorSubcoreMesh(core_axis_name="c", subcore_axis_name="s")
    count_kernel = functools.partial(
        pl.kernel,
        out_type=jax.ShapeDtypeStruct((N_PAD, CNT_W), jnp.float32),
        mesh=mesh,
        scratch_types=[
            pltpu.VMEM((CH,), jnp.int32),
            pltpu.VMEM((CH, CNT_W), jnp.float32),
            pltpu.VMEM_SHARED((N_PAD, CNT_W), jnp.float32),
        ],
    )(_count_body)
    agg_kernel = functools.partial(
        pl.kernel,
        out_type=jax.ShapeDtypeStruct((NQ * N_PAD, QD), jnp.float32),
        mesh=mesh,
        scratch_types=(
            [
                pltpu.VMEM((GCH,), jnp.int32),
                pltpu.VMEM((GCH,), jnp.int32),
                pltpu.VMEM((NBUF, CH), jnp.int32),
                pltpu.VMEM((NBUF, CH), jnp.int32),
            ]
            + [pltpu.VMEM((CH, QD), jnp.float32) for _ in range(NBUF)]
            + [pltpu.VMEM_SHARED((ACC_R, QD), jnp.float32)]
            + [pltpu.SemaphoreType.DMA for _ in range(4 + 2 * NBUF)]
        ),
    )(_agg_body)
    return count_kernel, agg_kernel


def _inproj_body(x_ref, w_ref, b_ref, o_ref):
    o_ref[...] = (
        jnp.dot(x_ref[...], w_ref[...], preferred_element_type=jnp.float32)
        + b_ref[...]
    )


def _lnwr_body(h_ref, g_ref, bt_ref, wr_ref, br_ref, hn_ref, self_ref):
    h = h_ref[...]
    mu = jnp.mean(h, axis=1, keepdims=True)
    var = jnp.mean((h - mu) ** 2, axis=1, keepdims=True)
    hn = (h - mu) * lax.rsqrt(var + 1e-5) * g_ref[...] + bt_ref[...]
    hn_ref[...] = hn.reshape(TR, NQ, QD).transpose(1, 0, 2)
    self_ref[...] = (
        jnp.dot(hn, wr_ref[...], preferred_element_type=jnp.float32)
        + br_ref[...]
    )


def _mix_body(agg_ref, cnt_ref, wl_ref, bl_ref, self_ref, h_ref, jk_ref,
              ho_ref, jko_ref):
    agg = agg_ref[...].transpose(1, 0, 2).reshape(TR, HID)
    scale = 1.0 / jnp.maximum(cnt_ref[:, :1], 1.0)
    z = (
        jnp.dot(agg * scale, wl_ref[...], preferred_element_type=jnp.float32)
        + bl_ref[...]
        + self_ref[...]
    )
    hnew = jnp.maximum(h_ref[...] + z, 0.0)
    ho_ref[...] = hnew
    jko_ref[...] = jnp.maximum(jk_ref[...], hnew)


def _outproj_body(jk_ref, w_ref, b_ref, o_ref):
    o_ref[...] = (
        jnp.dot(jk_ref[...], w_ref[...], preferred_element_type=jnp.float32)
        + b_ref[...]
    )


def _row_spec(w):
    return pl.BlockSpec((TR, w), lambda i: (i, 0))


def _full_spec(shape):
    return pl.BlockSpec(shape, lambda i: tuple(0 for _ in shape))


_inproj = pl.pallas_call(
    _inproj_body,
    grid=(GRID,),
    in_specs=[_row_spec(IN_DIM), _full_spec((IN_DIM, HID)),
              _full_spec((1, HID))],
    out_specs=_row_spec(HID),
    out_shape=jax.ShapeDtypeStruct((N_PAD, HID), jnp.float32),
)

_lnwr = pl.pallas_call(
    _lnwr_body,
    grid=(GRID,),
    in_specs=[_row_spec(HID), _full_spec((1, HID)), _full_spec((1, HID)),
              _full_spec((HID, HID)), _full_spec((1, HID))],
    out_specs=[
        pl.BlockSpec((NQ, TR, QD), lambda i: (0, i, 0)),
        _row_spec(HID),
    ],
    out_shape=[
        jax.ShapeDtypeStruct((NQ, N_PAD, QD), jnp.float32),
        jax.ShapeDtypeStruct((N_PAD, HID), jnp.float32),
    ],
)

_mix = pl.pallas_call(
    _mix_body,
    grid=(GRID,),
    in_specs=[
        pl.BlockSpec((NQ, TR, QD), lambda i: (0, i, 0)),
        _row_spec(CNT_W),
        _full_spec((HID, HID)),
        _full_spec((1, HID)),
        _row_spec(HID),
        _row_spec(HID),
        _row_spec(HID),
    ],
    out_specs=[_row_spec(HID), _row_spec(HID)],
    out_shape=[
        jax.ShapeDtypeStruct((N_PAD, HID), jnp.float32),
        jax.ShapeDtypeStruct((N_PAD, HID), jnp.float32),
    ],
)

_outproj = pl.pallas_call(
    _outproj_body,
    grid=(GRID,),
    in_specs=[_row_spec(HID), _full_spec((HID, OUT_DIM)),
              _full_spec((1, OUT_DIM))],
    out_specs=_row_spec(OUT_DIM),
    out_shape=jax.ShapeDtypeStruct((N_PAD, OUT_DIM), jnp.float32),
)


def kernel(x, edge_index, params):
    src = edge_index[0].astype(jnp.int32)
    dst = edge_index[1].astype(jnp.int32)
    dst4 = dst.reshape(NT * NGRP, NBUF, CH)
    x_p = jnp.pad(x, ((0, N_PAD - N), (0, 0)))

    p = params
    count_kernel, agg_kernel = _sc_kernels()
    h = _inproj(x_p, p["Win"], p["bin"].reshape(1, HID))
    cnt = count_kernel(dst)

    jk = jnp.zeros((N_PAD, HID), jnp.float32)
    for i in range(NUM_LAYERS):
        hn4, self_term = _lnwr(
            h,
            p["ln_g"][i].reshape(1, HID),
            p["ln_b"][i].reshape(1, HID),
            p["Wr"][i],
            p["br"][i].reshape(1, HID),
        )
        agg = agg_kernel(hn4.reshape(NQ * N_PAD, QD), src, dst4)
        h, jk = _mix(
            agg.reshape(NQ, N_PAD, QD),
            cnt,
            p["Wl"][i],
            p["bl"][i].reshape(1, HID),
            self_term,
            h,
            jk,
        )

    out = _outproj(jk, p["Wout"], p["bout"].reshape(1, OUT_DIM))
    return out[:N]


# speedup vs baseline: 1.2347x; 1.2347x over previous
import functools

import jax
import jax.numpy as jnp
from jax import lax
from jax.experimental import pallas as pl
from jax.experimental.pallas import tpu as pltpu
from jax.experimental.pallas import tpu_sc as plsc

N = 10000
E = 160000
IN_DIM = 256
HID = 512
OUT_DIM = 128
NUM_LAYERS = 4

N_PAD = 10240
TR = 512
GRID = N_PAD // TR

NQ = 4
QD = HID // NQ
NT = 16
EPT = E // NT
CH = 80
NCH = EPT // CH
RPT = N_PAD // NT
CNT_W = 128
EPT_C = E // (2 * NT)
NCHP = EPT_C // CH + 1


def _count_body(dst2_hbm, cnt_hbm, dst_all, ones_v, acc_sh, ssem):
    c = lax.axis_index("c")
    s = lax.axis_index("s")
    w = c * NT + s
    zeros16 = jnp.zeros((16,), jnp.float32)

    pltpu.sync_copy(dst2_hbm.at[w], dst_all)

    def _zero_row(i, _):
        for j in range(CNT_W // 16):
            ones_v[i, pl.ds(16 * j, 16)] = zeros16
        return 0

    lax.fori_loop(0, CH, _zero_row, 0)
    for k in range(RPT // CH):
        pltpu.sync_copy(ones_v, acc_sh.at[pl.ds(s * RPT + k * CH, CH)])

    def _one_row(i, _):
        for j in range(CNT_W // 16):
            ones_v[i, pl.ds(16 * j, 16)] = zeros16 + 1.0
        return 0

    lax.fori_loop(0, CH, _one_row, 0)
    plsc.subcore_barrier()

    GK = 2

    def _grp(g, _):
        for b in range(GK):
            pltpu.async_copy(ones_v, acc_sh.at[dst_all.at[GK * g + b]],
                             ssem, add=True)
        for b in range(GK):
            pltpu.make_async_copy(ones_v, acc_sh.at[pl.ds(0, CH)],
                                  ssem).wait()
        return 0

    lax.fori_loop(0, NCHP // GK, _grp, 0)
    pltpu.async_copy(ones_v, acc_sh.at[dst_all.at[NCHP - 1]], ssem, add=True)
    pltpu.make_async_copy(ones_v, acc_sh.at[pl.ds(0, CH)], ssem).wait()
    plsc.subcore_barrier()

    pltpu.sync_copy(acc_sh.at[pl.ds(s * RPT, RPT)],
                    cnt_hbm.at[pl.ds(c * N_PAD + s * RPT, RPT)])


def _agg_body(hn_hbm, src_hbm, dst2_hbm, out_hbm, src_flat, dst_all,
              buf0, buf1, acc_sh, sem0, sem1):
    c = lax.axis_index("c")
    s = lax.axis_index("s")
    zeros16 = jnp.zeros((16,), jnp.float32)
    NPAIR = NCH // 2

    pltpu.sync_copy(src_hbm.at[pl.ds(s * EPT, EPT)], src_flat)
    pltpu.sync_copy(dst2_hbm.at[s], dst_all)

    def _gather(j, buf, sem):
        return pltpu.async_copy(hn_hbm.at[src_flat.at[pl.ds(j * CH, CH)]],
                                buf, sem)

    def _wait(buf, sem):
        pltpu.make_async_copy(hn_hbm.at[pl.ds(0, CH)], buf, sem).wait()

    for qi in range(2):
        qoff = (2 * c * N_PAD) if qi == 0 else N_PAD

        def _adjust(i, _):
            src_flat[pl.ds(16 * i, 16)] = src_flat[pl.ds(16 * i, 16)] + qoff
            return 0

        lax.fori_loop(0, EPT // 16, _adjust, 0)

        def _zero_row(i, _):
            for j in range(QD // 16):
                buf0[i, pl.ds(16 * j, 16)] = zeros16
            return 0

        lax.fori_loop(0, CH, _zero_row, 0)
        for k in range(RPT // CH):
            pltpu.sync_copy(buf0, acc_sh.at[pl.ds(s * RPT + k * CH, CH)])
        plsc.subcore_barrier()

        _gather(0, buf0, sem0)

        def _pair(i, _):
            _gather(2 * i + 1, buf1, sem1)
            _wait(buf0, sem0)
            pltpu.sync_copy(buf0, acc_sh.at[dst_all.at[2 * i]], add=True)
            _gather(2 * i + 2, buf0, sem0)
            _wait(buf1, sem1)
            pltpu.sync_copy(buf1, acc_sh.at[dst_all.at[2 * i + 1]], add=True)
            return 0

        lax.fori_loop(0, NPAIR, _pair, 0)
        _wait(buf0, sem0)
        pltpu.sync_copy(buf0, acc_sh.at[dst_all.at[NCH - 1]], add=True)
        plsc.subcore_barrier()

        pltpu.sync_copy(acc_sh.at[pl.ds(s * RPT, RPT)],
                        out_hbm.at[pl.ds((2 * c + qi) * N_PAD + s * RPT, RPT)])


@functools.cache
def _sc_kernels():
    mesh = plsc.VectorSubcoreMesh(core_axis_name="c", subcore_axis_name="s")
    count_kernel = functools.partial(
        pl.kernel,
        out_type=jax.ShapeDtypeStruct((2 * N_PAD, CNT_W), jnp.float32),
        mesh=mesh,
        scratch_types=[
            pltpu.VMEM((NCHP, CH), jnp.int32),
            pltpu.VMEM((CH, CNT_W), jnp.float32),
            pltpu.VMEM_SHARED((N_PAD, CNT_W), jnp.float32),
            pltpu.SemaphoreType.DMA,
        ],
    )(_count_body)
    agg_kernel = functools.partial(
        pl.kernel,
        out_type=jax.ShapeDtypeStruct((NQ * N_PAD, QD), jnp.float32),
        mesh=mesh,
        scratch_types=[
            pltpu.VMEM((EPT,), jnp.int32),
            pltpu.VMEM((NCH, CH), jnp.int32),
            pltpu.VMEM((CH, QD), jnp.float32),
            pltpu.VMEM((CH, QD), jnp.float32),
            pltpu.VMEM_SHARED((N_PAD, QD), jnp.float32),
            pltpu.SemaphoreType.DMA,
            pltpu.SemaphoreType.DMA,
        ],
    )(_agg_body)
    return count_kernel, agg_kernel


def _ln_self(h, g, bt, wr, br):
    mu = jnp.mean(h, axis=1, keepdims=True)
    var = jnp.mean((h - mu) ** 2, axis=1, keepdims=True)
    hn = (h - mu) * lax.rsqrt(var + 1e-5) * g + bt
    self_t = jnp.dot(hn, wr, preferred_element_type=jnp.float32) + br
    return hn.reshape(TR, NQ, QD).transpose(1, 0, 2), self_t


def _mix(agg4, cnt2, wl, bl, self_t, h, jk):
    agg = agg4.transpose(1, 0, 2).reshape(TR, HID)
    scale = 1.0 / jnp.maximum(cnt2[0, :, :1] + cnt2[1, :, :1], 1.0)
    z = (jnp.dot(agg * scale, wl, preferred_element_type=jnp.float32)
         + bl + self_t)
    hnew = jnp.maximum(h + z, 0.0)
    return hnew, jnp.maximum(jk, hnew)


def _in_body(x_ref, win_ref, bin_ref, g_ref, bt_ref, wr_ref, br_ref,
             h_ref, hn_ref, self_ref):
    h = (jnp.dot(x_ref[...], win_ref[...], preferred_element_type=jnp.float32)
         + bin_ref[...])
    h_ref[...] = h
    hn_ref[...], self_ref[...] = _ln_self(h, g_ref[...], bt_ref[...],
                                          wr_ref[...], br_ref[...])


def _layer_body(agg_ref, cnt_ref, wl_ref, bl_ref, self_ref, h_ref, jk_ref,
                g_ref, bt_ref, wr_ref, br_ref,
                ho_ref, jko_ref, hn_ref, selfo_ref):
    hnew, jknew = _mix(agg_ref[...], cnt_ref[...], wl_ref[...], bl_ref[...],
                       self_ref[...], h_ref[...], jk_ref[...])
    ho_ref[...] = hnew
    jko_ref[...] = jknew
    hn_ref[...], selfo_ref[...] = _ln_self(hnew, g_ref[...], bt_ref[...],
                                           wr_ref[...], br_ref[...])


def _fin_body(agg_ref, cnt_ref, wl_ref, bl_ref, self_ref, h_ref, jk_ref,
              wout_ref, bout_ref, o_ref):
    _, jknew = _mix(agg_ref[...], cnt_ref[...], wl_ref[...], bl_ref[...],
                    self_ref[...], h_ref[...], jk_ref[...])
    o_ref[...] = (jnp.dot(jknew, wout_ref[...],
                          preferred_element_type=jnp.float32) + bout_ref[...])


def _row_spec(w):
    return pl.BlockSpec((TR, w), lambda i: (i, 0))


def _full_spec(shape):
    return pl.BlockSpec(shape, lambda i: tuple(0 for _ in shape))


_agg_spec = pl.BlockSpec((NQ, TR, QD), lambda i: (0, i, 0))
_cnt_spec = pl.BlockSpec((2, TR, CNT_W), lambda i: (0, i, 0))
_hn_out = jax.ShapeDtypeStruct((NQ, N_PAD, QD), jnp.float32)
_row_out = jax.ShapeDtypeStruct((N_PAD, HID), jnp.float32)

_in_call = pl.pallas_call(
    _in_body,
    grid=(GRID,),
    in_specs=[_row_spec(IN_DIM), _full_spec((IN_DIM, HID)),
              _full_spec((1, HID)), _full_spec((1, HID)), _full_spec((1, HID)),
              _full_spec((HID, HID)), _full_spec((1, HID))],
    out_specs=[_row_spec(HID), _agg_spec, _row_spec(HID)],
    out_shape=[_row_out, _hn_out, _row_out],
)

_layer_call = pl.pallas_call(
    _layer_body,
    grid=(GRID,),
    in_specs=[_agg_spec, _cnt_spec, _full_spec((HID, HID)),
              _full_spec((1, HID)), _row_spec(HID), _row_spec(HID),
              _row_spec(HID), _full_spec((1, HID)), _full_spec((1, HID)),
              _full_spec((HID, HID)), _full_spec((1, HID))],
    out_specs=[_row_spec(HID), _row_spec(HID), _agg_spec, _row_spec(HID)],
    out_shape=[_row_out, _row_out, _hn_out, _row_out],
)

_fin_call = pl.pallas_call(
    _fin_body,
    grid=(GRID,),
    in_specs=[_agg_spec, _cnt_spec, _full_spec((HID, HID)),
              _full_spec((1, HID)), _row_spec(HID), _row_spec(HID),
              _row_spec(HID), _full_spec((HID, OUT_DIM)),
              _full_spec((1, OUT_DIM))],
    out_specs=_row_spec(OUT_DIM),
    out_shape=jax.ShapeDtypeStruct((N_PAD, OUT_DIM), jnp.float32),
)


def kernel(x, edge_index, params):
    src = edge_index[0].astype(jnp.int32)
    dst = edge_index[1].astype(jnp.int32)
    dst3 = dst.reshape(NT, NCH, CH)
    dstc = jnp.pad(dst.reshape(2 * NT, EPT_C),
                   ((0, 0), (0, NCHP * CH - EPT_C)),
                   constant_values=N_PAD - 1).reshape(2 * NT, NCHP, CH)
    x_p = jnp.pad(x, ((0, N_PAD - N), (0, 0)))

    p = params
    count_kernel, agg_kernel = _sc_kernels()
    cnt = count_kernel(dstc).reshape(2, N_PAD, CNT_W)

    def b1(v):
        return v.reshape(1, -1)

    h, hn4, self_t = _in_call(x_p, p["Win"], b1(p["bin"]), b1(p["ln_g"][0]),
                              b1(p["ln_b"][0]), p["Wr"][0], b1(p["br"][0]))

    jk = jnp.zeros((N_PAD, HID), jnp.float32)
    for i in range(NUM_LAYERS - 1):
        agg = agg_kernel(hn4.reshape(NQ * N_PAD, QD), src, dst3)
        h, jk, hn4, self_t = _layer_call(
            agg.reshape(NQ, N_PAD, QD), cnt, p["Wl"][i], b1(p["bl"][i]),
            self_t, h, jk, b1(p["ln_g"][i + 1]), b1(p["ln_b"][i + 1]),
            p["Wr"][i + 1], b1(p["br"][i + 1]))

    agg = agg_kernel(hn4.reshape(NQ * N_PAD, QD), src, dst3)
    i = NUM_LAYERS - 1
    out = _fin_call(agg.reshape(NQ, N_PAD, QD), cnt, p["Wl"][i],
                    b1(p["bl"][i]), self_t, h, jk, p["Wout"], b1(p["bout"]))
    return out[:N]
